# Initial kernel scaffold; baseline (speedup 1.0000x reference)
import jax
import jax.numpy as jnp
from jax.experimental import pallas as pl


def kernel(x, edge_index, W_in, b_in, gW1, gb1, gW2, gb2, Wq, bq, Wk, bk, Wv, bv, Wo, bo, bn1_g, bn1_b, bn2_g, bn2_b, Wm1, bm1, Wm2, bm2, bn3_g, bn3_b, obn_g, obn_b, W_root, b_root, W_rel, b_rel):
    raise NotImplementedError("write your pallas kernel here")


# speedup vs baseline: 2.2811x; 2.2811x over previous
import functools

import jax
import jax.numpy as jnp
from jax import lax
from jax.experimental import pallas as pl
from jax.experimental.pallas import tpu as pltpu
from jax.experimental.pallas import tpu_sc as plsc

_N = 10000
_C = 128
_E = 320000
_H = 8
_DH = 16
_NPAD = 10240
_EPS = 1e-5
_BM = 1000
_BQ = 256
_SCALE = 0.25

_CH = 128
_NCHUNK = _E // _CH
_NTILES = 32
_PER = _NCHUNK // _NTILES
_REM = _NCHUNK - _PER * _NTILES


def _sc_scatter_kernel(h, edge_index, zeros_init):
    mesh = plsc.VectorSubcoreMesh(core_axis_name="c", subcore_axis_name="s")

    @functools.partial(
        pl.kernel,
        mesh=mesh,
        out_type=jax.ShapeDtypeStruct((2, _N, _C), jnp.float32),
        scratch_types=[
            pltpu.VMEM((_CH,), jnp.int32),
            pltpu.VMEM((_CH,), jnp.int32),
            pltpu.VMEM((_CH, _C), jnp.float32),
            pltpu.VMEM_SHARED((_N, _C), jnp.float32),
            pltpu.SemaphoreType.DMA,
        ],
    )
    def k(h_hbm, ei_hbm, z_hbm, out_hbm, src_v, dst_v, rows_v, acc, sem):
        c = lax.axis_index("c")
        s = lax.axis_index("s")
        w = s * 2 + c

        @pl.when(s == 0)
        def _():
            pltpu.sync_copy(z_hbm, acc)

        plsc.subcore_barrier()

        def chunk(ci):
            off = ci * _CH
            pltpu.sync_copy(ei_hbm.at[0, pl.ds(off, _CH)], src_v)
            pltpu.sync_copy(ei_hbm.at[1, pl.ds(off, _CH)], dst_v)
            pltpu.async_copy(h_hbm.at[src_v], rows_v, sem).wait()
            pltpu.sync_copy(rows_v, acc.at[dst_v], add=True)

        def body(i, carry):
            chunk(w * _PER + i)
            return carry

        lax.fori_loop(0, _PER, body, 0)

        @pl.when(w < _REM)
        def _():
            chunk(_NTILES * _PER + w)

        plsc.subcore_barrier()

        @pl.when(s == 0)
        def _():
            pltpu.sync_copy(acc, out_hbm.at[c])

    return k(h, edge_index, zeros_init)


def _scatter_partials(h, edge_index):
    zeros_init = jnp.zeros((_N, _C), jnp.float32)
    return _sc_scatter_kernel(h, edge_index, zeros_init)


def _k1_body(x_ref, wi, bi, wq, bq_, wk, bk_, wv, bv_,
             h_ref, q_ref, k_ref, v_ref):
    h = jnp.dot(x_ref[...], wi[...], preferred_element_type=jnp.float32) + bi[...]
    h_ref[...] = h
    q_ref[...] = jnp.dot(h, wq[...], preferred_element_type=jnp.float32) + bq_[...]
    k_ref[...] = jnp.dot(h, wk[...], preferred_element_type=jnp.float32) + bk_[...]
    v_ref[...] = jnp.dot(h, wv[...], preferred_element_type=jnp.float32) + bv_[...]


def _k1(x, WiT, bi, WqT, bq, WkT, bk, WvT, bv):
    row = pl.BlockSpec((_BM, _C), lambda i: (i, 0))
    full = pl.BlockSpec((_C, _C), lambda i: (0, 0))
    vec = pl.BlockSpec((1, _C), lambda i: (0, 0))
    out = jax.ShapeDtypeStruct((_N, _C), jnp.float32)
    return pl.pallas_call(
        _k1_body,
        grid=(_N // _BM,),
        in_specs=[row, full, vec, full, vec, full, vec, full, vec],
        out_specs=[row, row, row, row],
        out_shape=[out, out, out, out],
    )(x, WiT, bi, WqT, bq, WkT, bk, WvT, bv)


def _attn_body(qt_ref, kt_ref, vt_ref, ot_ref):
    qt = qt_ref[0]
    kt = kt_ref[0]
    vt = vt_ref[0]
    st = lax.dot_general(kt, qt, (((0,), (0,)), ((), ())),
                         preferred_element_type=jnp.float32)
    st = st * _SCALE
    ids = lax.broadcasted_iota(jnp.int32, (_NPAD, 1), 0)
    st = jnp.where(ids < _N, st, -1e30)
    m = jnp.max(st, axis=0, keepdims=True)
    p = jnp.exp(st - m)
    denom = jnp.sum(p, axis=0, keepdims=True)
    o = lax.dot_general(vt, p, (((1,), (0,)), ((), ())),
                        preferred_element_type=jnp.float32)
    ot_ref[0] = o / denom


def _attn(qT, kT, vT):
    qspec = pl.BlockSpec((1, _DH, _BQ), lambda h, j: (h, 0, j))
    kspec = pl.BlockSpec((1, _DH, _NPAD), lambda h, j: (h, 0, 0))
    return pl.pallas_call(
        _attn_body,
        grid=(_H, _NPAD // _BQ),
        in_specs=[qspec, kspec, kspec],
        out_specs=qspec,
        out_shape=jax.ShapeDtypeStruct((_H, _DH, _NPAD), jnp.float32),
    )(qT, kT, vT)


def _k3_body(h_ref, a0_ref, a1_ref, ac_ref, g1t, g1b, g2t, g2b, wot, bo_,
             t1_ref, t2_ref, s1_ref, q1_ref, s2_ref, q2_ref):
    i = pl.program_id(0)
    h = h_ref[...]
    z = h + a0_ref[...] + a1_ref[...]
    u = jnp.maximum(jnp.dot(z, g1t[...], preferred_element_type=jnp.float32)
                    + g1b[...], 0.0)
    t1 = jnp.dot(u, g2t[...], preferred_element_type=jnp.float32) + g2b[...] + h
    t2 = jnp.dot(ac_ref[...], wot[...], preferred_element_type=jnp.float32) \
        + bo_[...] + h
    t1_ref[...] = t1
    t2_ref[...] = t2

    @pl.when(i == 0)
    def _():
        s1_ref[...] = jnp.zeros_like(s1_ref)
        q1_ref[...] = jnp.zeros_like(q1_ref)
        s2_ref[...] = jnp.zeros_like(s2_ref)
        q2_ref[...] = jnp.zeros_like(q2_ref)

    s1_ref[...] += jnp.sum(t1, axis=0, keepdims=True)
    q1_ref[...] += jnp.sum(t1 * t1, axis=0, keepdims=True)
    s2_ref[...] += jnp.sum(t2, axis=0, keepdims=True)
    q2_ref[...] += jnp.sum(t2 * t2, axis=0, keepdims=True)


def _k3(h, a0, a1, ac, g1t, g1b, g2t, g2b, wot, bo):
    row = pl.BlockSpec((_BM, _C), lambda i: (i, 0))
    full = pl.BlockSpec((_C, _C), lambda i: (0, 0))
    vec = pl.BlockSpec((1, _C), lambda i: (0, 0))
    big = jax.ShapeDtypeStruct((_N, _C), jnp.float32)
    st = jax.ShapeDtypeStruct((1, _C), jnp.float32)
    return pl.pallas_call(
        _k3_body,
        grid=(_N // _BM,),
        in_specs=[row, row, row, row, full, vec, full, vec, full, vec],
        out_specs=[row, row, vec, vec, vec, vec],
        out_shape=[big, big, st, st, st, st],
    )(h, a0, a1, ac, g1t, g1b, g2t, g2b, wot, bo)


def _k4_body(t1_ref, t2_ref, s1, q1, s2, q2, g1, b1, g2, b2,
             wm1t, bm1_, wm2t, bm2_, t3_ref, s3_ref, q3_ref):
    i = pl.program_id(0)
    inv_n = 1.0 / _N
    mu1 = s1[...] * inv_n
    var1 = q1[...] * inv_n - mu1 * mu1
    sc1 = g1[...] * lax.rsqrt(var1 + _EPS)
    sh1 = b1[...] - mu1 * sc1
    mu2 = s2[...] * inv_n
    var2 = q2[...] * inv_n - mu2 * mu2
    sc2 = g2[...] * lax.rsqrt(var2 + _EPS)
    sh2 = b2[...] - mu2 * sc2
    out0 = t1_ref[...] * sc1 + sh1 + t2_ref[...] * sc2 + sh2
    mm = jnp.maximum(jnp.dot(out0, wm1t[...], preferred_element_type=jnp.float32)
                     + bm1_[...], 0.0)
    t3 = out0 + jnp.dot(mm, wm2t[...], preferred_element_type=jnp.float32) \
        + bm2_[...]
    t3_ref[...] = t3

    @pl.when(i == 0)
    def _():
        s3_ref[...] = jnp.zeros_like(s3_ref)
        q3_ref[...] = jnp.zeros_like(q3_ref)

    s3_ref[...] += jnp.sum(t3, axis=0, keepdims=True)
    q3_ref[...] += jnp.sum(t3 * t3, axis=0, keepdims=True)


def _k4(t1, t2, s1, q1, s2, q2, g1, b1, g2, b2, wm1t, bm1, wm2t, bm2):
    row = pl.BlockSpec((_BM, _C), lambda i: (i, 0))
    vec = pl.BlockSpec((1, _C), lambda i: (0, 0))
    vec2 = pl.BlockSpec((1, 2 * _C), lambda i: (0, 0))
    w1 = pl.BlockSpec((_C, 2 * _C), lambda i: (0, 0))
    w2 = pl.BlockSpec((2 * _C, _C), lambda i: (0, 0))
    big = jax.ShapeDtypeStruct((_N, _C), jnp.float32)
    st = jax.ShapeDtypeStruct((1, _C), jnp.float32)
    return pl.pallas_call(
        _k4_body,
        grid=(_N // _BM,),
        in_specs=[row, row, vec, vec, vec, vec, vec, vec, vec, vec,
                  w1, vec2, w2, vec],
        out_specs=[row, vec, vec],
        out_shape=[big, st, st],
    )(t1, t2, s1, q1, s2, q2, g1, b1, g2, b2, wm1t, bm1, wm2t, bm2)


def _k5_body(t3_ref, s3, q3, g3, b3, t4_ref, s4_ref, q4_ref):
    i = pl.program_id(0)
    inv_n = 1.0 / _N
    mu = s3[...] * inv_n
    var = q3[...] * inv_n - mu * mu
    sc = g3[...] * lax.rsqrt(var + _EPS)
    sh = b3[...] - mu * sc
    t4 = jnp.maximum(t3_ref[...] * sc + sh, 0.0)
    t4_ref[...] = t4

    @pl.when(i == 0)
    def _():
        s4_ref[...] = jnp.zeros_like(s4_ref)
        q4_ref[...] = jnp.zeros_like(q4_ref)

    s4_ref[...] += jnp.sum(t4, axis=0, keepdims=True)
    q4_ref[...] += jnp.sum(t4 * t4, axis=0, keepdims=True)


def _k5(t3, s3, q3, g3, b3):
    row = pl.BlockSpec((_BM, _C), lambda i: (i, 0))
    vec = pl.BlockSpec((1, _C), lambda i: (0, 0))
    big = jax.ShapeDtypeStruct((_N, _C), jnp.float32)
    st = jax.ShapeDtypeStruct((1, _C), jnp.float32)
    return pl.pallas_call(
        _k5_body,
        grid=(_N // _BM,),
        in_specs=[row, vec, vec, vec, vec],
        out_specs=[row, vec, vec],
        out_shape=[big, st, st],
    )(t3, s3, q3, g3, b3)


def _k6_body(t4_ref, s4, q4, g, b, wrt, brow, y_ref):
    inv_n = 1.0 / _N
    mu = s4[...] * inv_n
    var = q4[...] * inv_n - mu * mu
    sc = g[...] * lax.rsqrt(var + _EPS)
    sh = b[...] - mu * sc
    out2 = t4_ref[...] * sc + sh
    y_ref[...] = jnp.dot(out2, wrt[...], preferred_element_type=jnp.float32) \
        + brow[...]


def _k6(t4, s4, q4, g, b, wrt, brow):
    row = pl.BlockSpec((_BM, _C), lambda i: (i, 0))
    vec = pl.BlockSpec((1, _C), lambda i: (0, 0))
    full = pl.BlockSpec((_C, _C), lambda i: (0, 0))
    return pl.pallas_call(
        _k6_body,
        grid=(_N // _BM,),
        in_specs=[row, vec, vec, vec, vec, full, vec],
        out_specs=row,
        out_shape=jax.ShapeDtypeStruct((_N, _C), jnp.float32),
    )(t4, s4, q4, g, b, wrt, brow)


def kernel(x, edge_index, W_in, b_in, gW1, gb1, gW2, gb2, Wq, bq, Wk, bk,
           Wv, bv, Wo, bo, bn1_g, bn1_b, bn2_g, bn2_b, Wm1, bm1, Wm2, bm2,
           bn3_g, bn3_b, obn_g, obn_b, W_root, b_root, W_rel, b_rel):
    r = lambda t: t.reshape(1, -1)
    h, q, k, v = _k1(x, W_in.T, r(b_in), Wq.T, r(bq), Wk.T, r(bk),
                     Wv.T, r(bv))
    parts = _scatter_partials(h, edge_index)

    def t3d(a):
        a = a.reshape(_N, _H, _DH).transpose(1, 2, 0)
        return jnp.pad(a, ((0, 0), (0, 0), (0, _NPAD - _N)))

    aT = _attn(t3d(q), t3d(k), t3d(v))
    ac = aT.transpose(2, 0, 1).reshape(_NPAD, _C)[:_N]

    t1, t2, s1, q1, s2, q2 = _k3(h, parts[0], parts[1], ac,
                                 gW1.T, r(gb1), gW2.T, r(gb2), Wo.T, r(bo))
    t3, s3, q3 = _k4(t1, t2, s1, q1, s2, q2, r(bn1_g), r(bn1_b),
                     r(bn2_g), r(bn2_b), Wm1.T, r(bm1), Wm2.T, r(bm2))
    t4, s4, q4 = _k5(t3, s3, q3, r(bn3_g), r(bn3_b))
    wrt = jnp.pad((W_root + W_rel).T, ((0, 0), (0, _C - 40)))
    brow = jnp.pad(b_root + b_rel, (0, _C - 40)).reshape(1, -1)
    y = _k6(t4, s4, q4, r(obn_g), r(obn_b), wrt, brow)
    return y[:, :40]


# speedup vs baseline: 2.3414x; 1.0265x over previous
import functools

import jax
import jax.numpy as jnp
from jax import lax
from jax.experimental import pallas as pl
from jax.experimental.pallas import tpu as pltpu
from jax.experimental.pallas import tpu_sc as plsc

_N = 10000
_C = 128
_E = 320000
_H = 8
_DH = 16
_NPAD = 10240
_EPS = 1e-5
_BM = 1000
_BQ = 256
_SCALE = 0.25

_CH = 128
_NCHUNK = _E // _CH
_NTILES = 32
_PER = _NCHUNK // _NTILES
_REM = _NCHUNK - _PER * _NTILES


def _sc_scatter_kernel(h, edge_index, zeros_init):
    mesh = plsc.VectorSubcoreMesh(core_axis_name="c", subcore_axis_name="s")

    @functools.partial(
        pl.kernel,
        mesh=mesh,
        out_type=jax.ShapeDtypeStruct((2, _N, _C), jnp.float32),
        scratch_types=[
            pltpu.VMEM((_CH,), jnp.int32),
            pltpu.VMEM((_CH,), jnp.int32),
            pltpu.VMEM((_CH, _C), jnp.float32),
            pltpu.VMEM_SHARED((_N, _C), jnp.float32),
            pltpu.SemaphoreType.DMA,
        ],
    )
    def k(h_hbm, ei_hbm, z_hbm, out_hbm, src_v, dst_v, rows_v, acc, sem):
        c = lax.axis_index("c")
        s = lax.axis_index("s")
        w = s * 2 + c

        @pl.when(s == 0)
        def _():
            pltpu.sync_copy(z_hbm, acc)

        plsc.subcore_barrier()

        def chunk(ci):
            off = ci * _CH
            pltpu.sync_copy(ei_hbm.at[0, pl.ds(off, _CH)], src_v)
            pltpu.sync_copy(ei_hbm.at[1, pl.ds(off, _CH)], dst_v)
            pltpu.async_copy(h_hbm.at[src_v], rows_v, sem).wait()
            pltpu.sync_copy(rows_v, acc.at[dst_v], add=True)

        def body(i, carry):
            chunk(w * _PER + i)
            return carry

        lax.fori_loop(0, _PER, body, 0)

        @pl.when(w < _REM)
        def _():
            chunk(_NTILES * _PER + w)

        plsc.subcore_barrier()

        @pl.when(s == 0)
        def _():
            pltpu.sync_copy(acc, out_hbm.at[c])

    return k(h, edge_index, zeros_init)


def _scatter_partials(h, edge_index):
    zeros_init = jnp.zeros((_N, _C), jnp.float32)
    return _sc_scatter_kernel(h, edge_index, zeros_init)


def _k1_body(x_ref, wi, bi, wq, bq_, wk, bk_, wv, bv_,
             h_ref, q_ref, k_ref, v_ref):
    h = jnp.dot(x_ref[...], wi[...], preferred_element_type=jnp.float32) + bi[...]
    h_ref[...] = h
    q = (jnp.dot(h, wq[...], preferred_element_type=jnp.float32)
         + bq_[...]) * _SCALE
    q_ref[...] = q.astype(jnp.bfloat16)
    k_ref[...] = (jnp.dot(h, wk[...], preferred_element_type=jnp.float32)
                  + bk_[...]).astype(jnp.bfloat16)
    v_ref[...] = (jnp.dot(h, wv[...], preferred_element_type=jnp.float32)
                  + bv_[...]).astype(jnp.bfloat16)


def _k1(x, WiT, bi, WqT, bq, WkT, bk, WvT, bv):
    row = pl.BlockSpec((_BM, _C), lambda i: (i, 0))
    full = pl.BlockSpec((_C, _C), lambda i: (0, 0))
    vec = pl.BlockSpec((1, _C), lambda i: (0, 0))
    out = jax.ShapeDtypeStruct((_N, _C), jnp.float32)
    outb = jax.ShapeDtypeStruct((_N, _C), jnp.bfloat16)
    return pl.pallas_call(
        _k1_body,
        grid=(_N // _BM,),
        in_specs=[row, full, vec, full, vec, full, vec, full, vec],
        out_specs=[row, row, row, row],
        out_shape=[out, outb, outb, outb],
    )(x, WiT, bi, WqT, bq, WkT, bk, WvT, bv)


def _attn_body(qt_ref, kt_ref, vt_ref, ot_ref):
    qt = qt_ref[0]
    kt = kt_ref[0]
    vt = vt_ref[0]
    st = lax.dot_general(kt, qt, (((0,), (0,)), ((), ())),
                         preferred_element_type=jnp.float32)
    m = jnp.max(st, axis=0, keepdims=True)
    p = jnp.exp(st - m).astype(jnp.bfloat16)
    oe = lax.dot_general(vt, p, (((1,), (0,)), ((), ())),
                         preferred_element_type=jnp.float32)
    ot_ref[0] = oe[:_DH] / oe[_DH:_DH + 1]


def _attn(qT, kT, vTe):
    qspec = pl.BlockSpec((1, _DH, _BQ), lambda h, j: (h, 0, j))
    kspec = pl.BlockSpec((1, _DH, _NPAD), lambda h, j: (h, 0, 0))
    vspec = pl.BlockSpec((1, 2 * _DH, _NPAD), lambda h, j: (h, 0, 0))
    ospec = pl.BlockSpec((1, _DH, _BQ), lambda h, j: (h, 0, j))
    return pl.pallas_call(
        _attn_body,
        grid=(_H, _NPAD // _BQ),
        in_specs=[qspec, kspec, vspec],
        out_specs=ospec,
        out_shape=jax.ShapeDtypeStruct((_H, _DH, _NPAD), jnp.float32),
    )(qT, kT, vTe)


def _k3_body(h_ref, a0_ref, a1_ref, ac_ref, g1t, g1b, g2t, g2b, wot, bo_,
             t1_ref, t2_ref, s1_ref, q1_ref, s2_ref, q2_ref):
    i = pl.program_id(0)
    h = h_ref[...]
    z = h + a0_ref[...] + a1_ref[...]
    u = jnp.maximum(jnp.dot(z, g1t[...], preferred_element_type=jnp.float32)
                    + g1b[...], 0.0)
    t1 = jnp.dot(u, g2t[...], preferred_element_type=jnp.float32) + g2b[...] + h
    t2 = jnp.dot(ac_ref[...], wot[...], preferred_element_type=jnp.float32) \
        + bo_[...] + h
    t1_ref[...] = t1
    t2_ref[...] = t2

    @pl.when(i == 0)
    def _():
        s1_ref[...] = jnp.zeros_like(s1_ref)
        q1_ref[...] = jnp.zeros_like(q1_ref)
        s2_ref[...] = jnp.zeros_like(s2_ref)
        q2_ref[...] = jnp.zeros_like(q2_ref)

    s1_ref[...] += jnp.sum(t1, axis=0, keepdims=True)
    q1_ref[...] += jnp.sum(t1 * t1, axis=0, keepdims=True)
    s2_ref[...] += jnp.sum(t2, axis=0, keepdims=True)
    q2_ref[...] += jnp.sum(t2 * t2, axis=0, keepdims=True)


def _k3(h, a0, a1, ac, g1t, g1b, g2t, g2b, wot, bo):
    row = pl.BlockSpec((_BM, _C), lambda i: (i, 0))
    full = pl.BlockSpec((_C, _C), lambda i: (0, 0))
    vec = pl.BlockSpec((1, _C), lambda i: (0, 0))
    big = jax.ShapeDtypeStruct((_N, _C), jnp.float32)
    st = jax.ShapeDtypeStruct((1, _C), jnp.float32)
    return pl.pallas_call(
        _k3_body,
        grid=(_N // _BM,),
        in_specs=[row, row, row, row, full, vec, full, vec, full, vec],
        out_specs=[row, row, vec, vec, vec, vec],
        out_shape=[big, big, st, st, st, st],
    )(h, a0, a1, ac, g1t, g1b, g2t, g2b, wot, bo)


def _k4_body(t1_ref, t2_ref, s1, q1, s2, q2, g1, b1, g2, b2,
             wm1t, bm1_, wm2t, bm2_, t3_ref, s3_ref, q3_ref):
    i = pl.program_id(0)
    inv_n = 1.0 / _N
    mu1 = s1[...] * inv_n
    var1 = q1[...] * inv_n - mu1 * mu1
    sc1 = g1[...] * lax.rsqrt(var1 + _EPS)
    sh1 = b1[...] - mu1 * sc1
    mu2 = s2[...] * inv_n
    var2 = q2[...] * inv_n - mu2 * mu2
    sc2 = g2[...] * lax.rsqrt(var2 + _EPS)
    sh2 = b2[...] - mu2 * sc2
    out0 = t1_ref[...] * sc1 + sh1 + t2_ref[...] * sc2 + sh2
    mm = jnp.maximum(jnp.dot(out0, wm1t[...], preferred_element_type=jnp.float32)
                     + bm1_[...], 0.0)
    t3 = out0 + jnp.dot(mm, wm2t[...], preferred_element_type=jnp.float32) \
        + bm2_[...]
    t3_ref[...] = t3

    @pl.when(i == 0)
    def _():
        s3_ref[...] = jnp.zeros_like(s3_ref)
        q3_ref[...] = jnp.zeros_like(q3_ref)

    s3_ref[...] += jnp.sum(t3, axis=0, keepdims=True)
    q3_ref[...] += jnp.sum(t3 * t3, axis=0, keepdims=True)


def _k4(t1, t2, s1, q1, s2, q2, g1, b1, g2, b2, wm1t, bm1, wm2t, bm2):
    row = pl.BlockSpec((_BM, _C), lambda i: (i, 0))
    vec = pl.BlockSpec((1, _C), lambda i: (0, 0))
    vec2 = pl.BlockSpec((1, 2 * _C), lambda i: (0, 0))
    w1 = pl.BlockSpec((_C, 2 * _C), lambda i: (0, 0))
    w2 = pl.BlockSpec((2 * _C, _C), lambda i: (0, 0))
    big = jax.ShapeDtypeStruct((_N, _C), jnp.float32)
    st = jax.ShapeDtypeStruct((1, _C), jnp.float32)
    return pl.pallas_call(
        _k4_body,
        grid=(_N // _BM,),
        in_specs=[row, row, vec, vec, vec, vec, vec, vec, vec, vec,
                  w1, vec2, w2, vec],
        out_specs=[row, vec, vec],
        out_shape=[big, st, st],
    )(t1, t2, s1, q1, s2, q2, g1, b1, g2, b2, wm1t, bm1, wm2t, bm2)


def _k5_body(t3_ref, s3, q3, g3, b3, t4_ref, s4_ref, q4_ref):
    i = pl.program_id(0)
    inv_n = 1.0 / _N
    mu = s3[...] * inv_n
    var = q3[...] * inv_n - mu * mu
    sc = g3[...] * lax.rsqrt(var + _EPS)
    sh = b3[...] - mu * sc
    t4 = jnp.maximum(t3_ref[...] * sc + sh, 0.0)
    t4_ref[...] = t4

    @pl.when(i == 0)
    def _():
        s4_ref[...] = jnp.zeros_like(s4_ref)
        q4_ref[...] = jnp.zeros_like(q4_ref)

    s4_ref[...] += jnp.sum(t4, axis=0, keepdims=True)
    q4_ref[...] += jnp.sum(t4 * t4, axis=0, keepdims=True)


def _k5(t3, s3, q3, g3, b3):
    row = pl.BlockSpec((_BM, _C), lambda i: (i, 0))
    vec = pl.BlockSpec((1, _C), lambda i: (0, 0))
    big = jax.ShapeDtypeStruct((_N, _C), jnp.float32)
    st = jax.ShapeDtypeStruct((1, _C), jnp.float32)
    return pl.pallas_call(
        _k5_body,
        grid=(_N // _BM,),
        in_specs=[row, vec, vec, vec, vec],
        out_specs=[row, vec, vec],
        out_shape=[big, st, st],
    )(t3, s3, q3, g3, b3)


def _k6_body(t4_ref, s4, q4, g, b, wrt, brow, y_ref):
    inv_n = 1.0 / _N
    mu = s4[...] * inv_n
    var = q4[...] * inv_n - mu * mu
    sc = g[...] * lax.rsqrt(var + _EPS)
    sh = b[...] - mu * sc
    out2 = t4_ref[...] * sc + sh
    y_ref[...] = jnp.dot(out2, wrt[...], preferred_element_type=jnp.float32) \
        + brow[...]


def _k6(t4, s4, q4, g, b, wrt, brow):
    row = pl.BlockSpec((_BM, _C), lambda i: (i, 0))
    vec = pl.BlockSpec((1, _C), lambda i: (0, 0))
    full = pl.BlockSpec((_C, _C), lambda i: (0, 0))
    return pl.pallas_call(
        _k6_body,
        grid=(_N // _BM,),
        in_specs=[row, vec, vec, vec, vec, full, vec],
        out_specs=row,
        out_shape=jax.ShapeDtypeStruct((_N, _C), jnp.float32),
    )(t4, s4, q4, g, b, wrt, brow)


def kernel(x, edge_index, W_in, b_in, gW1, gb1, gW2, gb2, Wq, bq, Wk, bk,
           Wv, bv, Wo, bo, bn1_g, bn1_b, bn2_g, bn2_b, Wm1, bm1, Wm2, bm2,
           bn3_g, bn3_b, obn_g, obn_b, W_root, b_root, W_rel, b_rel):
    r = lambda t: t.reshape(1, -1)
    h, q, k, v = _k1(x, W_in.T, r(b_in), Wq.T, r(bq), Wk.T, r(bk),
                     Wv.T, r(bv))
    parts = _scatter_partials(h, edge_index)

    def t3d(a):
        a = a.reshape(_N, _H, _DH).transpose(1, 2, 0)
        return jnp.pad(a, ((0, 0), (0, 0), (0, _NPAD - _N)))

    vTe = jnp.concatenate(
        [t3d(v),
         jnp.broadcast_to((jnp.arange(_NPAD) < _N).astype(jnp.bfloat16),
                          (_H, 1, _NPAD)),
         jnp.zeros((_H, _DH - 1, _NPAD), jnp.bfloat16)], axis=1)
    aT = _attn(t3d(q), t3d(k), vTe)
    ac = aT.transpose(2, 0, 1).reshape(_NPAD, _C)[:_N]

    t1, t2, s1, q1, s2, q2 = _k3(h, parts[0], parts[1], ac,
                                 gW1.T, r(gb1), gW2.T, r(gb2), Wo.T, r(bo))
    t3, s3, q3 = _k4(t1, t2, s1, q1, s2, q2, r(bn1_g), r(bn1_b),
                     r(bn2_g), r(bn2_b), Wm1.T, r(bm1), Wm2.T, r(bm2))
    t4, s4, q4 = _k5(t3, s3, q3, r(bn3_g), r(bn3_b))
    wrt = jnp.pad((W_root + W_rel).T, ((0, 0), (0, _C - 40)))
    brow = jnp.pad(b_root + b_rel, (0, _C - 40)).reshape(1, -1)
    y = _k6(t4, s4, q4, r(obn_g), r(obn_b), wrt, brow)
    return y[:, :40]


# speedup vs baseline: 2.3546x; 1.0056x over previous
import functools

import jax
import jax.numpy as jnp
from jax import lax
from jax.experimental import pallas as pl
from jax.experimental.pallas import tpu as pltpu
from jax.experimental.pallas import tpu_sc as plsc

_N = 10000
_C = 128
_E = 320000
_H = 8
_DH = 16
_NPAD = 10240
_EPS = 1e-5
_BM = 1000
_BQ = 256
_SCALE = 0.25 * 1.4426950408889634

_CH = 128
_NCHUNK = _E // _CH
_NTILES = 32
_PER = _NCHUNK // _NTILES
_REM = _NCHUNK - _PER * _NTILES


def _sc_scatter_kernel(h, edge_index, zeros_init):
    mesh = plsc.VectorSubcoreMesh(core_axis_name="c", subcore_axis_name="s")

    @functools.partial(
        pl.kernel,
        mesh=mesh,
        out_type=jax.ShapeDtypeStruct((2, _N, _C), jnp.float32),
        scratch_types=[
            pltpu.VMEM((_CH,), jnp.int32),
            pltpu.VMEM((_CH,), jnp.int32),
            pltpu.VMEM((_CH, _C), jnp.float32),
            pltpu.VMEM_SHARED((_N, _C), jnp.float32),
            pltpu.SemaphoreType.DMA,
        ],
    )
    def k(h_hbm, ei_hbm, z_hbm, out_hbm, src_v, dst_v, rows_v, acc, sem):
        c = lax.axis_index("c")
        s = lax.axis_index("s")
        w = s * 2 + c

        @pl.when(s == 0)
        def _():
            pltpu.sync_copy(z_hbm, acc)

        plsc.subcore_barrier()

        def chunk(ci):
            off = ci * _CH
            pltpu.sync_copy(ei_hbm.at[0, pl.ds(off, _CH)], src_v)
            pltpu.sync_copy(ei_hbm.at[1, pl.ds(off, _CH)], dst_v)
            pltpu.async_copy(h_hbm.at[src_v], rows_v, sem).wait()
            pltpu.sync_copy(rows_v, acc.at[dst_v], add=True)

        def body(i, carry):
            chunk(w * _PER + i)
            return carry

        lax.fori_loop(0, _PER, body, 0)

        @pl.when(w < _REM)
        def _():
            chunk(_NTILES * _PER + w)

        plsc.subcore_barrier()

        @pl.when(s == 0)
        def _():
            pltpu.sync_copy(acc, out_hbm.at[c])

    return k(h, edge_index, zeros_init)


def _scatter_partials(h, edge_index):
    zeros_init = jnp.zeros((_N, _C), jnp.float32)
    return _sc_scatter_kernel(h, edge_index, zeros_init)


def _k1_body(x_ref, wi, bi, wq, bq_, wk, bk_, wv, bv_,
             h_ref, q_ref, k_ref, v_ref):
    h = jnp.dot(x_ref[...], wi[...], preferred_element_type=jnp.float32) + bi[...]
    h_ref[...] = h
    q = (jnp.dot(h, wq[...], preferred_element_type=jnp.float32)
         + bq_[...]) * _SCALE
    q_ref[...] = q.astype(jnp.bfloat16)
    k_ref[...] = (jnp.dot(h, wk[...], preferred_element_type=jnp.float32)
                  + bk_[...]).astype(jnp.bfloat16)
    v_ref[...] = (jnp.dot(h, wv[...], preferred_element_type=jnp.float32)
                  + bv_[...]).astype(jnp.bfloat16)


def _k1(x, WiT, bi, WqT, bq, WkT, bk, WvT, bv):
    row = pl.BlockSpec((_BM, _C), lambda i: (i, 0))
    full = pl.BlockSpec((_C, _C), lambda i: (0, 0))
    vec = pl.BlockSpec((1, _C), lambda i: (0, 0))
    out = jax.ShapeDtypeStruct((_N, _C), jnp.float32)
    outb = jax.ShapeDtypeStruct((_N, _C), jnp.bfloat16)
    return pl.pallas_call(
        _k1_body,
        grid=(_N // _BM,),
        in_specs=[row, full, vec, full, vec, full, vec, full, vec],
        out_specs=[row, row, row, row],
        out_shape=[out, outb, outb, outb],
    )(x, WiT, bi, WqT, bq, WkT, bk, WvT, bv)


def _attn_body(qt_ref, kt_ref, vt_ref, ot_ref):
    qt = qt_ref[0]
    kt = kt_ref[0]
    vt = vt_ref[0]
    st = lax.dot_general(kt, qt, (((0,), (0,)), ((), ())),
                         preferred_element_type=jnp.float32)
    m = jnp.max(st, axis=0, keepdims=True)
    p = jnp.exp2(st - m).astype(jnp.bfloat16)
    oe = lax.dot_general(vt, p, (((1,), (0,)), ((), ())),
                         preferred_element_type=jnp.float32)
    ot_ref[0] = oe[:_DH] / oe[_DH:_DH + 1]


def _attn(qT, kT, vTe):
    qspec = pl.BlockSpec((1, _DH, _BQ), lambda h, j: (h, 0, j))
    kspec = pl.BlockSpec((1, _DH, _NPAD), lambda h, j: (h, 0, 0))
    vspec = pl.BlockSpec((1, 2 * _DH, _NPAD), lambda h, j: (h, 0, 0))
    ospec = pl.BlockSpec((1, _DH, _BQ), lambda h, j: (h, 0, j))
    return pl.pallas_call(
        _attn_body,
        grid=(_H, _NPAD // _BQ),
        in_specs=[qspec, kspec, vspec],
        out_specs=ospec,
        out_shape=jax.ShapeDtypeStruct((_H, _DH, _NPAD), jnp.float32),
    )(qT, kT, vTe)


def _k3_body(h_ref, a0_ref, a1_ref, ac_ref, g1t, g1b, g2t, g2b, wot, bo_,
             t1_ref, t2_ref, s1_ref, q1_ref, s2_ref, q2_ref):
    i = pl.program_id(0)
    h = h_ref[...]
    z = h + a0_ref[...] + a1_ref[...]
    u = jnp.maximum(jnp.dot(z, g1t[...], preferred_element_type=jnp.float32)
                    + g1b[...], 0.0)
    t1 = jnp.dot(u, g2t[...], preferred_element_type=jnp.float32) + g2b[...] + h
    t2 = jnp.dot(ac_ref[...], wot[...], preferred_element_type=jnp.float32) \
        + bo_[...] + h
    t1_ref[...] = t1
    t2_ref[...] = t2

    @pl.when(i == 0)
    def _():
        s1_ref[...] = jnp.zeros_like(s1_ref)
        q1_ref[...] = jnp.zeros_like(q1_ref)
        s2_ref[...] = jnp.zeros_like(s2_ref)
        q2_ref[...] = jnp.zeros_like(q2_ref)

    s1_ref[...] += jnp.sum(t1, axis=0, keepdims=True)
    q1_ref[...] += jnp.sum(t1 * t1, axis=0, keepdims=True)
    s2_ref[...] += jnp.sum(t2, axis=0, keepdims=True)
    q2_ref[...] += jnp.sum(t2 * t2, axis=0, keepdims=True)


def _k3(h, a0, a1, ac, g1t, g1b, g2t, g2b, wot, bo):
    row = pl.BlockSpec((_BM, _C), lambda i: (i, 0))
    full = pl.BlockSpec((_C, _C), lambda i: (0, 0))
    vec = pl.BlockSpec((1, _C), lambda i: (0, 0))
    big = jax.ShapeDtypeStruct((_N, _C), jnp.float32)
    st = jax.ShapeDtypeStruct((1, _C), jnp.float32)
    return pl.pallas_call(
        _k3_body,
        grid=(_N // _BM,),
        in_specs=[row, row, row, row, full, vec, full, vec, full, vec],
        out_specs=[row, row, vec, vec, vec, vec],
        out_shape=[big, big, st, st, st, st],
    )(h, a0, a1, ac, g1t, g1b, g2t, g2b, wot, bo)


def _k4_body(t1_ref, t2_ref, s1, q1, s2, q2, g1, b1, g2, b2,
             wm1t, bm1_, wm2t, bm2_, t3_ref, s3_ref, q3_ref):
    i = pl.program_id(0)
    inv_n = 1.0 / _N
    mu1 = s1[...] * inv_n
    var1 = q1[...] * inv_n - mu1 * mu1
    sc1 = g1[...] * lax.rsqrt(var1 + _EPS)
    sh1 = b1[...] - mu1 * sc1
    mu2 = s2[...] * inv_n
    var2 = q2[...] * inv_n - mu2 * mu2
    sc2 = g2[...] * lax.rsqrt(var2 + _EPS)
    sh2 = b2[...] - mu2 * sc2
    out0 = t1_ref[...] * sc1 + sh1 + t2_ref[...] * sc2 + sh2
    mm = jnp.maximum(jnp.dot(out0, wm1t[...], preferred_element_type=jnp.float32)
                     + bm1_[...], 0.0)
    t3 = out0 + jnp.dot(mm, wm2t[...], preferred_element_type=jnp.float32) \
        + bm2_[...]
    t3_ref[...] = t3

    @pl.when(i == 0)
    def _():
        s3_ref[...] = jnp.zeros_like(s3_ref)
        q3_ref[...] = jnp.zeros_like(q3_ref)

    s3_ref[...] += jnp.sum(t3, axis=0, keepdims=True)
    q3_ref[...] += jnp.sum(t3 * t3, axis=0, keepdims=True)


def _k4(t1, t2, s1, q1, s2, q2, g1, b1, g2, b2, wm1t, bm1, wm2t, bm2):
    row = pl.BlockSpec((_BM, _C), lambda i: (i, 0))
    vec = pl.BlockSpec((1, _C), lambda i: (0, 0))
    vec2 = pl.BlockSpec((1, 2 * _C), lambda i: (0, 0))
    w1 = pl.BlockSpec((_C, 2 * _C), lambda i: (0, 0))
    w2 = pl.BlockSpec((2 * _C, _C), lambda i: (0, 0))
    big = jax.ShapeDtypeStruct((_N, _C), jnp.float32)
    st = jax.ShapeDtypeStruct((1, _C), jnp.float32)
    return pl.pallas_call(
        _k4_body,
        grid=(_N // _BM,),
        in_specs=[row, row, vec, vec, vec, vec, vec, vec, vec, vec,
                  w1, vec2, w2, vec],
        out_specs=[row, vec, vec],
        out_shape=[big, st, st],
    )(t1, t2, s1, q1, s2, q2, g1, b1, g2, b2, wm1t, bm1, wm2t, bm2)


def _k5_body(t3_ref, s3, q3, g3, b3, t4_ref, s4_ref, q4_ref):
    i = pl.program_id(0)
    inv_n = 1.0 / _N
    mu = s3[...] * inv_n
    var = q3[...] * inv_n - mu * mu
    sc = g3[...] * lax.rsqrt(var + _EPS)
    sh = b3[...] - mu * sc
    t4 = jnp.maximum(t3_ref[...] * sc + sh, 0.0)
    t4_ref[...] = t4

    @pl.when(i == 0)
    def _():
        s4_ref[...] = jnp.zeros_like(s4_ref)
        q4_ref[...] = jnp.zeros_like(q4_ref)

    s4_ref[...] += jnp.sum(t4, axis=0, keepdims=True)
    q4_ref[...] += jnp.sum(t4 * t4, axis=0, keepdims=True)


def _k5(t3, s3, q3, g3, b3):
    row = pl.BlockSpec((_BM, _C), lambda i: (i, 0))
    vec = pl.BlockSpec((1, _C), lambda i: (0, 0))
    big = jax.ShapeDtypeStruct((_N, _C), jnp.float32)
    st = jax.ShapeDtypeStruct((1, _C), jnp.float32)
    return pl.pallas_call(
        _k5_body,
        grid=(_N // _BM,),
        in_specs=[row, vec, vec, vec, vec],
        out_specs=[row, vec, vec],
        out_shape=[big, st, st],
    )(t3, s3, q3, g3, b3)


def _k6_body(t4_ref, s4, q4, g, b, wrt, brow, y_ref):
    inv_n = 1.0 / _N
    mu = s4[...] * inv_n
    var = q4[...] * inv_n - mu * mu
    sc = g[...] * lax.rsqrt(var + _EPS)
    sh = b[...] - mu * sc
    out2 = t4_ref[...] * sc + sh
    y_ref[...] = jnp.dot(out2, wrt[...], preferred_element_type=jnp.float32) \
        + brow[...]


def _k6(t4, s4, q4, g, b, wrt, brow):
    row = pl.BlockSpec((_BM, _C), lambda i: (i, 0))
    vec = pl.BlockSpec((1, _C), lambda i: (0, 0))
    full = pl.BlockSpec((_C, _C), lambda i: (0, 0))
    return pl.pallas_call(
        _k6_body,
        grid=(_N // _BM,),
        in_specs=[row, vec, vec, vec, vec, full, vec],
        out_specs=row,
        out_shape=jax.ShapeDtypeStruct((_N, _C), jnp.float32),
    )(t4, s4, q4, g, b, wrt, brow)


def kernel(x, edge_index, W_in, b_in, gW1, gb1, gW2, gb2, Wq, bq, Wk, bk,
           Wv, bv, Wo, bo, bn1_g, bn1_b, bn2_g, bn2_b, Wm1, bm1, Wm2, bm2,
           bn3_g, bn3_b, obn_g, obn_b, W_root, b_root, W_rel, b_rel):
    r = lambda t: t.reshape(1, -1)
    h, q, k, v = _k1(x, W_in.T, r(b_in), Wq.T, r(bq), Wk.T, r(bk),
                     Wv.T, r(bv))
    parts = _scatter_partials(h, edge_index)

    def t3d(a):
        a = a.reshape(_N, _H, _DH).transpose(1, 2, 0)
        return jnp.pad(a, ((0, 0), (0, 0), (0, _NPAD - _N)))

    vTe = jnp.concatenate(
        [t3d(v),
         jnp.broadcast_to((jnp.arange(_NPAD) < _N).astype(jnp.bfloat16),
                          (_H, 1, _NPAD)),
         jnp.zeros((_H, _DH - 1, _NPAD), jnp.bfloat16)], axis=1)
    aT = _attn(t3d(q), t3d(k), vTe)
    ac = aT.transpose(2, 0, 1).reshape(_NPAD, _C)[:_N]

    t1, t2, s1, q1, s2, q2 = _k3(h, parts[0], parts[1], ac,
                                 gW1.T, r(gb1), gW2.T, r(gb2), Wo.T, r(bo))
    t3, s3, q3 = _k4(t1, t2, s1, q1, s2, q2, r(bn1_g), r(bn1_b),
                     r(bn2_g), r(bn2_b), Wm1.T, r(bm1), Wm2.T, r(bm2))
    t4, s4, q4 = _k5(t3, s3, q3, r(bn3_g), r(bn3_b))
    wrt = jnp.pad((W_root + W_rel).T, ((0, 0), (0, _C - 40)))
    brow = jnp.pad(b_root + b_rel, (0, _C - 40)).reshape(1, -1)
    y = _k6(t4, s4, q4, r(obn_g), r(obn_b), wrt, brow)
    return y[:, :40]


# speedup vs baseline: 3.1174x; 1.3240x over previous
import functools

import jax
import jax.numpy as jnp
from jax import lax
from jax.experimental import pallas as pl
from jax.experimental.pallas import tpu as pltpu
from jax.experimental.pallas import tpu_sc as plsc

_N = 10000
_C = 128
_E = 320000
_H = 8
_DH = 16
_NPAD = 10240
_EPS = 1e-5
_BM = 1000
_BQ = 512
_SCALE = 0.25 * 1.4426950408889634

_CH = 128
_NCHUNK = _E // _CH
_NTILES = 32
_PER = _NCHUNK // _NTILES
_REM = _NCHUNK - _PER * _NTILES


def _sc_scatter_kernel(h, edge_index, zeros_init):
    mesh = plsc.VectorSubcoreMesh(core_axis_name="c", subcore_axis_name="s")

    @functools.partial(
        pl.kernel,
        mesh=mesh,
        out_type=jax.ShapeDtypeStruct((2, _N, _C), jnp.float32),
        scratch_types=[
            pltpu.VMEM((_CH,), jnp.int32),
            pltpu.VMEM((_CH,), jnp.int32),
            pltpu.VMEM((_CH, _C), jnp.float32),
            pltpu.VMEM_SHARED((_N, _C), jnp.float32),
            pltpu.SemaphoreType.DMA,
        ],
    )
    def k(h_hbm, ei_hbm, z_hbm, out_hbm, src_v, dst_v, rows_v, acc, sem):
        c = lax.axis_index("c")
        s = lax.axis_index("s")
        w = s * 2 + c

        @pl.when(s == 0)
        def _():
            pltpu.sync_copy(z_hbm, acc)

        plsc.subcore_barrier()

        def chunk(ci):
            off = ci * _CH
            pltpu.sync_copy(ei_hbm.at[0, pl.ds(off, _CH)], src_v)
            pltpu.sync_copy(ei_hbm.at[1, pl.ds(off, _CH)], dst_v)
            pltpu.async_copy(h_hbm.at[src_v], rows_v, sem).wait()
            pltpu.sync_copy(rows_v, acc.at[dst_v], add=True)

        def body(i, carry):
            chunk(w * _PER + i)
            return carry

        lax.fori_loop(0, _PER, body, 0)

        @pl.when(w < _REM)
        def _():
            chunk(_NTILES * _PER + w)

        plsc.subcore_barrier()

        @pl.when(s == 0)
        def _():
            pltpu.sync_copy(acc, out_hbm.at[c])

    return k(h, edge_index, zeros_init)


def _scatter_partials(h, edge_index):
    zeros_init = jnp.zeros((_N, _C), jnp.float32)
    return _sc_scatter_kernel(h, edge_index, zeros_init)


def _k1_body(x_ref, wi, bi, wq, bq_, wk, bk_, wv, bv_,
             h_ref, q_ref, k_ref, v_ref):
    h = jnp.dot(x_ref[...], wi[...], preferred_element_type=jnp.float32) + bi[...]
    h_ref[...] = h
    q = (jnp.dot(h, wq[...], preferred_element_type=jnp.float32)
         + bq_[...]) * _SCALE
    q_ref[...] = q.astype(jnp.bfloat16)
    k_ref[...] = (jnp.dot(h, wk[...], preferred_element_type=jnp.float32)
                  + bk_[...]).astype(jnp.bfloat16)
    v_ref[...] = (jnp.dot(h, wv[...], preferred_element_type=jnp.float32)
                  + bv_[...]).astype(jnp.bfloat16)


def _k1(x, WiT, bi, WqT, bq, WkT, bk, WvT, bv):
    row = pl.BlockSpec((_BM, _C), lambda i: (i, 0))
    full = pl.BlockSpec((_C, _C), lambda i: (0, 0))
    vec = pl.BlockSpec((1, _C), lambda i: (0, 0))
    out = jax.ShapeDtypeStruct((_N, _C), jnp.float32)
    outb = jax.ShapeDtypeStruct((_N, _C), jnp.bfloat16)
    return pl.pallas_call(
        _k1_body,
        grid=(_N // _BM,),
        in_specs=[row, full, vec, full, vec, full, vec, full, vec],
        out_specs=[row, row, row, row],
        out_shape=[out, outb, outb, outb],
    )(x, WiT, bi, WqT, bq, WkT, bk, WvT, bv)


def _attn_body(qt_ref, kt_ref, vt_ref, ot_ref):
    qt = qt_ref[0]
    kt = kt_ref[0]
    vt = vt_ref[0]
    st = lax.dot_general(kt, qt, (((0,), (0,)), ((), ())),
                         preferred_element_type=jnp.float32)
    m = jnp.max(st, axis=0, keepdims=True)
    p = jnp.exp2(st - m).astype(jnp.bfloat16)
    oe = lax.dot_general(vt, p, (((1,), (0,)), ((), ())),
                         preferred_element_type=jnp.float32)
    ot_ref[0] = oe[:_DH] / oe[_DH:_DH + 1]


def _attn(qT, kT, vTe):
    qspec = pl.BlockSpec((1, _DH, _BQ), lambda h, j: (h, 0, j))
    kspec = pl.BlockSpec((1, _DH, _NPAD), lambda h, j: (h, 0, 0))
    vspec = pl.BlockSpec((1, 2 * _DH, _NPAD), lambda h, j: (h, 0, 0))
    ospec = pl.BlockSpec((1, _DH, _BQ), lambda h, j: (h, 0, j))
    return pl.pallas_call(
        _attn_body,
        grid=(_H, _NPAD // _BQ),
        in_specs=[qspec, kspec, vspec],
        out_specs=ospec,
        out_shape=jax.ShapeDtypeStruct((_H, _DH, _NPAD), jnp.float32),
    )(qT, kT, vTe)


def _k3_body(h_ref, a0_ref, a1_ref, ac_ref, g1t, g1b, g2t, g2b, wot, bo_,
             t1_ref, t2_ref, s1_ref, q1_ref, s2_ref, q2_ref):
    i = pl.program_id(0)
    h = h_ref[...]
    z = h + a0_ref[...] + a1_ref[...]
    u = jnp.maximum(jnp.dot(z, g1t[...], preferred_element_type=jnp.float32)
                    + g1b[...], 0.0)
    t1 = jnp.dot(u, g2t[...], preferred_element_type=jnp.float32) + g2b[...] + h
    t2 = jnp.dot(ac_ref[...], wot[...], preferred_element_type=jnp.float32) \
        + bo_[...] + h
    t1_ref[...] = t1
    t2_ref[...] = t2

    @pl.when(i == 0)
    def _():
        s1_ref[...] = jnp.zeros_like(s1_ref)
        q1_ref[...] = jnp.zeros_like(q1_ref)
        s2_ref[...] = jnp.zeros_like(s2_ref)
        q2_ref[...] = jnp.zeros_like(q2_ref)

    s1_ref[...] += jnp.sum(t1, axis=0, keepdims=True)
    q1_ref[...] += jnp.sum(t1 * t1, axis=0, keepdims=True)
    s2_ref[...] += jnp.sum(t2, axis=0, keepdims=True)
    q2_ref[...] += jnp.sum(t2 * t2, axis=0, keepdims=True)


def _k3(h, a0, a1, ac, g1t, g1b, g2t, g2b, wot, bo):
    row = pl.BlockSpec((_BM, _C), lambda i: (i, 0))
    full = pl.BlockSpec((_C, _C), lambda i: (0, 0))
    vec = pl.BlockSpec((1, _C), lambda i: (0, 0))
    big = jax.ShapeDtypeStruct((_N, _C), jnp.float32)
    st = jax.ShapeDtypeStruct((1, _C), jnp.float32)
    return pl.pallas_call(
        _k3_body,
        grid=(_N // _BM,),
        in_specs=[row, row, row, row, full, vec, full, vec, full, vec],
        out_specs=[row, row, vec, vec, vec, vec],
        out_shape=[big, big, st, st, st, st],
    )(h, a0, a1, ac, g1t, g1b, g2t, g2b, wot, bo)


def _k4_body(t1_ref, t2_ref, s1, q1, s2, q2, g1, b1, g2, b2,
             wm1t, bm1_, wm2t, bm2_, t3_ref, s3_ref, q3_ref):
    i = pl.program_id(0)
    inv_n = 1.0 / _N
    mu1 = s1[...] * inv_n
    var1 = q1[...] * inv_n - mu1 * mu1
    sc1 = g1[...] * lax.rsqrt(var1 + _EPS)
    sh1 = b1[...] - mu1 * sc1
    mu2 = s2[...] * inv_n
    var2 = q2[...] * inv_n - mu2 * mu2
    sc2 = g2[...] * lax.rsqrt(var2 + _EPS)
    sh2 = b2[...] - mu2 * sc2
    out0 = t1_ref[...] * sc1 + sh1 + t2_ref[...] * sc2 + sh2
    mm = jnp.maximum(jnp.dot(out0, wm1t[...], preferred_element_type=jnp.float32)
                     + bm1_[...], 0.0)
    t3 = out0 + jnp.dot(mm, wm2t[...], preferred_element_type=jnp.float32) \
        + bm2_[...]
    t3_ref[...] = t3

    @pl.when(i == 0)
    def _():
        s3_ref[...] = jnp.zeros_like(s3_ref)
        q3_ref[...] = jnp.zeros_like(q3_ref)

    s3_ref[...] += jnp.sum(t3, axis=0, keepdims=True)
    q3_ref[...] += jnp.sum(t3 * t3, axis=0, keepdims=True)


def _k4(t1, t2, s1, q1, s2, q2, g1, b1, g2, b2, wm1t, bm1, wm2t, bm2):
    row = pl.BlockSpec((_BM, _C), lambda i: (i, 0))
    vec = pl.BlockSpec((1, _C), lambda i: (0, 0))
    vec2 = pl.BlockSpec((1, 2 * _C), lambda i: (0, 0))
    w1 = pl.BlockSpec((_C, 2 * _C), lambda i: (0, 0))
    w2 = pl.BlockSpec((2 * _C, _C), lambda i: (0, 0))
    big = jax.ShapeDtypeStruct((_N, _C), jnp.float32)
    st = jax.ShapeDtypeStruct((1, _C), jnp.float32)
    return pl.pallas_call(
        _k4_body,
        grid=(_N // _BM,),
        in_specs=[row, row, vec, vec, vec, vec, vec, vec, vec, vec,
                  w1, vec2, w2, vec],
        out_specs=[row, vec, vec],
        out_shape=[big, st, st],
    )(t1, t2, s1, q1, s2, q2, g1, b1, g2, b2, wm1t, bm1, wm2t, bm2)


def _k5_body(t3_ref, s3, q3, g3, b3, t4_ref, s4_ref, q4_ref):
    i = pl.program_id(0)
    inv_n = 1.0 / _N
    mu = s3[...] * inv_n
    var = q3[...] * inv_n - mu * mu
    sc = g3[...] * lax.rsqrt(var + _EPS)
    sh = b3[...] - mu * sc
    t4 = jnp.maximum(t3_ref[...] * sc + sh, 0.0)
    t4_ref[...] = t4

    @pl.when(i == 0)
    def _():
        s4_ref[...] = jnp.zeros_like(s4_ref)
        q4_ref[...] = jnp.zeros_like(q4_ref)

    s4_ref[...] += jnp.sum(t4, axis=0, keepdims=True)
    q4_ref[...] += jnp.sum(t4 * t4, axis=0, keepdims=True)


def _k5(t3, s3, q3, g3, b3):
    row = pl.BlockSpec((_BM, _C), lambda i: (i, 0))
    vec = pl.BlockSpec((1, _C), lambda i: (0, 0))
    big = jax.ShapeDtypeStruct((_N, _C), jnp.float32)
    st = jax.ShapeDtypeStruct((1, _C), jnp.float32)
    return pl.pallas_call(
        _k5_body,
        grid=(_N // _BM,),
        in_specs=[row, vec, vec, vec, vec],
        out_specs=[row, vec, vec],
        out_shape=[big, st, st],
    )(t3, s3, q3, g3, b3)


def _k6_body(t4_ref, s4, q4, g, b, wrt, brow, y_ref):
    inv_n = 1.0 / _N
    mu = s4[...] * inv_n
    var = q4[...] * inv_n - mu * mu
    sc = g[...] * lax.rsqrt(var + _EPS)
    sh = b[...] - mu * sc
    out2 = t4_ref[...] * sc + sh
    y_ref[...] = jnp.dot(out2, wrt[...], preferred_element_type=jnp.float32) \
        + brow[...]


def _k6(t4, s4, q4, g, b, wrt, brow):
    row = pl.BlockSpec((_BM, _C), lambda i: (i, 0))
    vec = pl.BlockSpec((1, _C), lambda i: (0, 0))
    full = pl.BlockSpec((_C, _C), lambda i: (0, 0))
    return pl.pallas_call(
        _k6_body,
        grid=(_N // _BM,),
        in_specs=[row, vec, vec, vec, vec, full, vec],
        out_specs=row,
        out_shape=jax.ShapeDtypeStruct((_N, _C), jnp.float32),
    )(t4, s4, q4, g, b, wrt, brow)


def kernel(x, edge_index, W_in, b_in, gW1, gb1, gW2, gb2, Wq, bq, Wk, bk,
           Wv, bv, Wo, bo, bn1_g, bn1_b, bn2_g, bn2_b, Wm1, bm1, Wm2, bm2,
           bn3_g, bn3_b, obn_g, obn_b, W_root, b_root, W_rel, b_rel):
    r = lambda t: t.reshape(1, -1)
    h, q, k, v = _k1(x, W_in.T, r(b_in), Wq.T, r(bq), Wk.T, r(bk),
                     Wv.T, r(bv))
    parts = _scatter_partials(h, edge_index)

    def t3d(a):
        a = a.reshape(_N, _H, _DH).transpose(1, 2, 0)
        return jnp.pad(a, ((0, 0), (0, 0), (0, _NPAD - _N)))

    vTe = jnp.concatenate(
        [t3d(v),
         jnp.broadcast_to((jnp.arange(_NPAD) < _N).astype(jnp.bfloat16),
                          (_H, 1, _NPAD)),
         jnp.zeros((_H, _DH - 1, _NPAD), jnp.bfloat16)], axis=1)
    aT = _attn(t3d(q), t3d(k), vTe)
    ac = aT.transpose(2, 0, 1).reshape(_NPAD, _C)[:_N]

    t1, t2, s1, q1, s2, q2 = _k3(h, parts[0], parts[1], ac,
                                 gW1.T, r(gb1), gW2.T, r(gb2), Wo.T, r(bo))
    t3, s3, q3 = _k4(t1, t2, s1, q1, s2, q2, r(bn1_g), r(bn1_b),
                     r(bn2_g), r(bn2_b), Wm1.T, r(bm1), Wm2.T, r(bm2))
    t4, s4, q4 = _k5(t3, s3, q3, r(bn3_g), r(bn3_b))
    wrt = jnp.pad((W_root + W_rel).T, ((0, 0), (0, _C - 40)))
    brow = jnp.pad(b_root + b_rel, (0, _C - 40)).reshape(1, -1)
    y = _k6(t4, s4, q4, r(obn_g), r(obn_b), wrt, brow)
    return y[:, :40]


# speedup vs baseline: 4.5362x; 1.4551x over previous
import functools

import jax
import jax.numpy as jnp
from jax import lax
from jax.experimental import pallas as pl
from jax.experimental.pallas import tpu as pltpu
from jax.experimental.pallas import tpu_sc as plsc

_N = 10000
_C = 128
_E = 320000
_H = 8
_DH = 16
_NPAD = 10240
_EPS = 1e-5
_BM = 1000
_BQ = 512
_SCALE = 0.25 * 1.4426950408889634

_CH = 128
_NCHUNK = _E // _CH
_NTILES = 32
_PER = _NCHUNK // _NTILES
_REM = _NCHUNK - _PER * _NTILES


def _sc_scatter_kernel(h, edge_index, zeros_init):
    mesh = plsc.VectorSubcoreMesh(core_axis_name="c", subcore_axis_name="s")

    @functools.partial(
        pl.kernel,
        mesh=mesh,
        out_type=jax.ShapeDtypeStruct((2, _N, _C), jnp.float32),
        scratch_types=[
            pltpu.VMEM((_CH,), jnp.int32),
            pltpu.VMEM((_CH,), jnp.int32),
            pltpu.VMEM((_CH, _C), jnp.float32),
            pltpu.VMEM_SHARED((_N, _C), jnp.float32),
            pltpu.SemaphoreType.DMA,
        ],
    )
    def k(h_hbm, ei_hbm, z_hbm, out_hbm, src_v, dst_v, rows_v, acc, sem):
        c = lax.axis_index("c")
        s = lax.axis_index("s")
        w = s * 2 + c

        @pl.when(s == 0)
        def _():
            pltpu.sync_copy(z_hbm, acc)

        plsc.subcore_barrier()

        def chunk(ci):
            off = ci * _CH
            pltpu.sync_copy(ei_hbm.at[0, pl.ds(off, _CH)], src_v)
            pltpu.sync_copy(ei_hbm.at[1, pl.ds(off, _CH)], dst_v)
            pltpu.async_copy(h_hbm.at[src_v], rows_v, sem).wait()
            pltpu.sync_copy(rows_v, acc.at[dst_v], add=True)

        def body(i, carry):
            chunk(w * _PER + i)
            return carry

        lax.fori_loop(0, _PER, body, 0)

        @pl.when(w < _REM)
        def _():
            chunk(_NTILES * _PER + w)

        plsc.subcore_barrier()

        @pl.when(s == 0)
        def _():
            pltpu.sync_copy(acc, out_hbm.at[c])

    return k(h, edge_index, zeros_init)


def _scatter_partials(h, edge_index):
    zeros_init = jnp.zeros((_N, _C), jnp.float32)
    return _sc_scatter_kernel(h, edge_index, zeros_init)


def _k1_body(x_ref, wi, bi, wq, bq_, wk, bk_, wv, bv_, hm,
             h_ref, q_ref, k_ref, v_ref, kn_ref):
    i = pl.program_id(0)
    h = jnp.dot(x_ref[...], wi[...], preferred_element_type=jnp.float32) + bi[...]
    h_ref[...] = h
    q = (jnp.dot(h, wq[...], preferred_element_type=jnp.float32)
         + bq_[...]) * _SCALE
    q_ref[...] = q.astype(jnp.bfloat16)
    k = jnp.dot(h, wk[...], preferred_element_type=jnp.float32) + bk_[...]
    k_ref[...] = k.astype(jnp.bfloat16)
    v_ref[...] = (jnp.dot(h, wv[...], preferred_element_type=jnp.float32)
                  + bv_[...]).astype(jnp.bfloat16)
    nsq = jnp.dot(k * k, hm[...], preferred_element_type=jnp.float32)
    cur = jnp.max(nsq, axis=0, keepdims=True)

    @pl.when(i == 0)
    def _():
        kn_ref[...] = jnp.zeros_like(kn_ref)

    kn_ref[...] = jnp.maximum(kn_ref[...], cur)


def _k1(x, WiT, bi, WqT, bq, WkT, bk, WvT, bv, hm):
    row = pl.BlockSpec((_BM, _C), lambda i: (i, 0))
    full = pl.BlockSpec((_C, _C), lambda i: (0, 0))
    vec = pl.BlockSpec((1, _C), lambda i: (0, 0))
    out = jax.ShapeDtypeStruct((_N, _C), jnp.float32)
    outb = jax.ShapeDtypeStruct((_N, _C), jnp.bfloat16)
    st = jax.ShapeDtypeStruct((1, _C), jnp.float32)
    return pl.pallas_call(
        _k1_body,
        grid=(_N // _BM,),
        in_specs=[row, full, vec, full, vec, full, vec, full, vec, full],
        out_specs=[row, row, row, row, vec],
        out_shape=[out, outb, outb, outb, st],
    )(x, WiT, bi, WqT, bq, WkT, bk, WvT, bv, hm)


def _attn_body(qt_ref, kt_ref, vt_ref, kn_ref, ot_ref):
    qt = qt_ref[0]
    kt = kt_ref[0]
    vt = vt_ref[0]
    st = lax.dot_general(kt, qt, (((0,), (0,)), ((), ())),
                         preferred_element_type=jnp.float32)
    qf = qt.astype(jnp.float32)
    qq = jnp.sum(qf * qf, axis=0, keepdims=True)
    b = jnp.sqrt(qq * kn_ref[0, 0, 0])
    p = jnp.exp2(st - b).astype(jnp.bfloat16)
    oe = lax.dot_general(vt, p, (((1,), (0,)), ((), ())),
                         preferred_element_type=jnp.float32)
    ot_ref[0] = oe[:_DH] / oe[_DH:_DH + 1]


def _attn(qT, kT, vTe, knb):
    qspec = pl.BlockSpec((1, _DH, _BQ), lambda h, j: (h, 0, j))
    kspec = pl.BlockSpec((1, _DH, _NPAD), lambda h, j: (h, 0, 0))
    vspec = pl.BlockSpec((1, 2 * _DH, _NPAD), lambda h, j: (h, 0, 0))
    nspec = pl.BlockSpec((1, 8, _C), lambda h, j: (h, 0, 0))
    ospec = pl.BlockSpec((1, _DH, _BQ), lambda h, j: (h, 0, j))
    return pl.pallas_call(
        _attn_body,
        grid=(_H, _NPAD // _BQ),
        in_specs=[qspec, kspec, vspec, nspec],
        out_specs=ospec,
        out_shape=jax.ShapeDtypeStruct((_H, _DH, _NPAD), jnp.float32),
    )(qT, kT, vTe, knb)


def _k3_body(h_ref, a0_ref, a1_ref, ac_ref, g1t, g1b, g2t, g2b, wot, bo_,
             t1_ref, t2_ref, s1_ref, q1_ref, s2_ref, q2_ref):
    i = pl.program_id(0)
    h = h_ref[...]
    z = h + a0_ref[...] + a1_ref[...]
    u = jnp.maximum(jnp.dot(z, g1t[...], preferred_element_type=jnp.float32)
                    + g1b[...], 0.0)
    t1 = jnp.dot(u, g2t[...], preferred_element_type=jnp.float32) + g2b[...] + h
    t2 = jnp.dot(ac_ref[...], wot[...], preferred_element_type=jnp.float32) \
        + bo_[...] + h
    t1_ref[...] = t1
    t2_ref[...] = t2

    @pl.when(i == 0)
    def _():
        s1_ref[...] = jnp.zeros_like(s1_ref)
        q1_ref[...] = jnp.zeros_like(q1_ref)
        s2_ref[...] = jnp.zeros_like(s2_ref)
        q2_ref[...] = jnp.zeros_like(q2_ref)

    s1_ref[...] += jnp.sum(t1, axis=0, keepdims=True)
    q1_ref[...] += jnp.sum(t1 * t1, axis=0, keepdims=True)
    s2_ref[...] += jnp.sum(t2, axis=0, keepdims=True)
    q2_ref[...] += jnp.sum(t2 * t2, axis=0, keepdims=True)


def _k3(h, a0, a1, ac, g1t, g1b, g2t, g2b, wot, bo):
    row = pl.BlockSpec((_BM, _C), lambda i: (i, 0))
    full = pl.BlockSpec((_C, _C), lambda i: (0, 0))
    vec = pl.BlockSpec((1, _C), lambda i: (0, 0))
    big = jax.ShapeDtypeStruct((_N, _C), jnp.float32)
    st = jax.ShapeDtypeStruct((1, _C), jnp.float32)
    return pl.pallas_call(
        _k3_body,
        grid=(_N // _BM,),
        in_specs=[row, row, row, row, full, vec, full, vec, full, vec],
        out_specs=[row, row, vec, vec, vec, vec],
        out_shape=[big, big, st, st, st, st],
    )(h, a0, a1, ac, g1t, g1b, g2t, g2b, wot, bo)


def _k4_body(t1_ref, t2_ref, s1, q1, s2, q2, g1, b1, g2, b2,
             wm1t, bm1_, wm2t, bm2_, t3_ref, s3_ref, q3_ref):
    i = pl.program_id(0)
    inv_n = 1.0 / _N
    mu1 = s1[...] * inv_n
    var1 = q1[...] * inv_n - mu1 * mu1
    sc1 = g1[...] * lax.rsqrt(var1 + _EPS)
    sh1 = b1[...] - mu1 * sc1
    mu2 = s2[...] * inv_n
    var2 = q2[...] * inv_n - mu2 * mu2
    sc2 = g2[...] * lax.rsqrt(var2 + _EPS)
    sh2 = b2[...] - mu2 * sc2
    out0 = t1_ref[...] * sc1 + sh1 + t2_ref[...] * sc2 + sh2
    mm = jnp.maximum(jnp.dot(out0, wm1t[...], preferred_element_type=jnp.float32)
                     + bm1_[...], 0.0)
    t3 = out0 + jnp.dot(mm, wm2t[...], preferred_element_type=jnp.float32) \
        + bm2_[...]
    t3_ref[...] = t3

    @pl.when(i == 0)
    def _():
        s3_ref[...] = jnp.zeros_like(s3_ref)
        q3_ref[...] = jnp.zeros_like(q3_ref)

    s3_ref[...] += jnp.sum(t3, axis=0, keepdims=True)
    q3_ref[...] += jnp.sum(t3 * t3, axis=0, keepdims=True)


def _k4(t1, t2, s1, q1, s2, q2, g1, b1, g2, b2, wm1t, bm1, wm2t, bm2):
    row = pl.BlockSpec((_BM, _C), lambda i: (i, 0))
    vec = pl.BlockSpec((1, _C), lambda i: (0, 0))
    vec2 = pl.BlockSpec((1, 2 * _C), lambda i: (0, 0))
    w1 = pl.BlockSpec((_C, 2 * _C), lambda i: (0, 0))
    w2 = pl.BlockSpec((2 * _C, _C), lambda i: (0, 0))
    big = jax.ShapeDtypeStruct((_N, _C), jnp.float32)
    st = jax.ShapeDtypeStruct((1, _C), jnp.float32)
    return pl.pallas_call(
        _k4_body,
        grid=(_N // _BM,),
        in_specs=[row, row, vec, vec, vec, vec, vec, vec, vec, vec,
                  w1, vec2, w2, vec],
        out_specs=[row, vec, vec],
        out_shape=[big, st, st],
    )(t1, t2, s1, q1, s2, q2, g1, b1, g2, b2, wm1t, bm1, wm2t, bm2)


def _k5_body(t3_ref, s3, q3, g3, b3, t4_ref, s4_ref, q4_ref):
    i = pl.program_id(0)
    inv_n = 1.0 / _N
    mu = s3[...] * inv_n
    var = q3[...] * inv_n - mu * mu
    sc = g3[...] * lax.rsqrt(var + _EPS)
    sh = b3[...] - mu * sc
    t4 = jnp.maximum(t3_ref[...] * sc + sh, 0.0)
    t4_ref[...] = t4

    @pl.when(i == 0)
    def _():
        s4_ref[...] = jnp.zeros_like(s4_ref)
        q4_ref[...] = jnp.zeros_like(q4_ref)

    s4_ref[...] += jnp.sum(t4, axis=0, keepdims=True)
    q4_ref[...] += jnp.sum(t4 * t4, axis=0, keepdims=True)


def _k5(t3, s3, q3, g3, b3):
    row = pl.BlockSpec((_BM, _C), lambda i: (i, 0))
    vec = pl.BlockSpec((1, _C), lambda i: (0, 0))
    big = jax.ShapeDtypeStruct((_N, _C), jnp.float32)
    st = jax.ShapeDtypeStruct((1, _C), jnp.float32)
    return pl.pallas_call(
        _k5_body,
        grid=(_N // _BM,),
        in_specs=[row, vec, vec, vec, vec],
        out_specs=[row, vec, vec],
        out_shape=[big, st, st],
    )(t3, s3, q3, g3, b3)


def _k6_body(t4_ref, s4, q4, g, b, wrt, brow, y_ref):
    inv_n = 1.0 / _N
    mu = s4[...] * inv_n
    var = q4[...] * inv_n - mu * mu
    sc = g[...] * lax.rsqrt(var + _EPS)
    sh = b[...] - mu * sc
    out2 = t4_ref[...] * sc + sh
    y_ref[...] = jnp.dot(out2, wrt[...], preferred_element_type=jnp.float32) \
        + brow[...]


def _k6(t4, s4, q4, g, b, wrt, brow):
    row = pl.BlockSpec((_BM, _C), lambda i: (i, 0))
    vec = pl.BlockSpec((1, _C), lambda i: (0, 0))
    full = pl.BlockSpec((_C, _C), lambda i: (0, 0))
    return pl.pallas_call(
        _k6_body,
        grid=(_N // _BM,),
        in_specs=[row, vec, vec, vec, vec, full, vec],
        out_specs=row,
        out_shape=jax.ShapeDtypeStruct((_N, _C), jnp.float32),
    )(t4, s4, q4, g, b, wrt, brow)


def kernel(x, edge_index, W_in, b_in, gW1, gb1, gW2, gb2, Wq, bq, Wk, bk,
           Wv, bv, Wo, bo, bn1_g, bn1_b, bn2_g, bn2_b, Wm1, bm1, Wm2, bm2,
           bn3_g, bn3_b, obn_g, obn_b, W_root, b_root, W_rel, b_rel):
    r = lambda t: t.reshape(1, -1)
    hm = (jnp.arange(_C)[:, None] // _DH
          == jnp.arange(_C)[None, :]).astype(jnp.float32)
    h, q, k, v, kn = _k1(x, W_in.T, r(b_in), Wq.T, r(bq), Wk.T, r(bk),
                         Wv.T, r(bv), hm)
    parts = _scatter_partials(h, edge_index)

    def t3d(a):
        a = a.reshape(_N, _H, _DH).transpose(1, 2, 0)
        return jnp.pad(a, ((0, 0), (0, 0), (0, _NPAD - _N)))

    vTe = jnp.concatenate(
        [t3d(v),
         jnp.broadcast_to((jnp.arange(_NPAD) < _N).astype(jnp.bfloat16),
                          (_H, 1, _NPAD)),
         jnp.zeros((_H, _DH - 1, _NPAD), jnp.bfloat16)], axis=1)
    knb = jnp.broadcast_to(kn[0, :_H, None, None], (_H, 8, _C))
    aT = _attn(t3d(q), t3d(k), vTe, knb)
    ac = aT.transpose(2, 0, 1).reshape(_NPAD, _C)[:_N]

    t1, t2, s1, q1, s2, q2 = _k3(h, parts[0], parts[1], ac,
                                 gW1.T, r(gb1), gW2.T, r(gb2), Wo.T, r(bo))
    t3, s3, q3 = _k4(t1, t2, s1, q1, s2, q2, r(bn1_g), r(bn1_b),
                     r(bn2_g), r(bn2_b), Wm1.T, r(bm1), Wm2.T, r(bm2))
    t4, s4, q4 = _k5(t3, s3, q3, r(bn3_g), r(bn3_b))
    wrt = jnp.pad((W_root + W_rel).T, ((0, 0), (0, _C - 40)))
    brow = jnp.pad(b_root + b_rel, (0, _C - 40)).reshape(1, -1)
    y = _k6(t4, s4, q4, r(obn_g), r(obn_b), wrt, brow)
    return y[:, :40]
